# baseline (speedup 1.0000x reference)
import functools

import jax
import jax.numpy as jnp
from jax import lax
from jax.experimental import pallas as pl
from jax.experimental.pallas import tpu as pltpu
from jax.experimental.pallas import tpu_sc as plsc

_N = 8192
_B = 8
_K = 20
_RT = 256
_CT = 256


def _knn_body(se_ref, cb_ref, xr_ref, btr_ref, sqr_ref, xc_ref, btc_ref,
              sqc_ref, idx_ref, strip_ref, cmin_ref, *, d):
    t = pl.program_id(0)
    c0 = cb_ref[t, 0]
    c1 = cb_ref[t, 1]
    nt = strip_ref.shape[0] // _CT

    rr = xr_ref[...]
    sqr = sqr_ref[...]
    btr = btr_ref[...]

    inf = jnp.float32(jnp.inf)
    big = jnp.int32(2 * _N)
    cmin_ref[...] = jnp.full(cmin_ref.shape, inf)

    def dist_tile(c, _):
        cc = xc_ref[pl.ds(c * _CT, _CT), :]
        g = jax.lax.dot_general(cc, rr, (((1,), (1,)), ((), ())),
                                preferred_element_type=jnp.float32)
        sqc = sqc_ref[pl.ds(c * _CT, _CT), :]
        d2 = (sqc + sqr) - 2.0 * g
        btc = btc_ref[pl.ds(c * _CT, _CT), :]
        d2 = jnp.where(btc != btr, inf, d2)
        strip_ref[pl.ds(c * _CT, _CT), :] = d2
        cmin_ref[pl.ds(c, 1), :] = jnp.min(d2, axis=0, keepdims=True)
        return 0

    jax.lax.fori_loop(c0, c1, dist_tile, 0)

    s_row = jnp.zeros((1, _RT), jnp.int32)
    e_row = jnp.zeros((1, _RT), jnp.int32)
    for b in range(_B):
        s_row = jnp.where(btr == b, se_ref[0, b], s_row)
        e_row = jnp.where(btr == b, se_ref[1, b], e_row)
    nvalid = e_row - s_row

    iota0 = jax.lax.broadcasted_iota(jnp.int32, (_CT, _RT), 0)

    for k in range(_K):
        cm = cmin_ref[...]
        m = jnp.min(cm, axis=0, keepdims=True)
        tc = jnp.full((1, _RT), big, jnp.int32)
        for c in range(nt):
            tc = jnp.minimum(
                tc, jnp.where(cm[c:c + 1, :] == m, c, big))

        def pickpass(c, idx):
            rowsel = tc == c

            def do_tile():
                tile = strip_ref[pl.ds(c * _CT, _CT), :]
                eq = (tile == m) & rowsel
                jstar = jnp.min(jnp.where(eq, iota0, big), axis=0,
                                keepdims=True)
                newtile = jnp.where(iota0 == jstar, inf, tile)
                strip_ref[pl.ds(c * _CT, _CT), :] = newtile
                cmin_ref[pl.ds(c, 1), :] = jnp.min(newtile, axis=0,
                                                   keepdims=True)
                return jnp.where(rowsel, c * _CT + jstar, idx)

            return jax.lax.cond(jnp.any(rowsel), do_tile, lambda: idx)

        idx = jax.lax.fori_loop(c0, c1, pickpass, jnp.full((1, _RT), big))

        p = k - nvalid
        idxfix = jnp.where(p < s_row, p, e_row + (p - s_row))
        idx = jnp.where(m == inf, idxfix, idx)
        idx_ref[k:k + 1, :] = idx


def _knn(x, btc2, btr2, sq, se, cb, d):
    n = x.shape[0]
    grid_spec = pltpu.PrefetchScalarGridSpec(
        num_scalar_prefetch=2,
        grid=(n // _RT,),
        in_specs=[
            pl.BlockSpec((_RT, d), lambda t, se, cb: (t, 0)),
            pl.BlockSpec((1, _RT), lambda t, se, cb: (0, t)),
            pl.BlockSpec((1, _RT), lambda t, se, cb: (0, t)),
            pl.BlockSpec((n, d), lambda t, se, cb: (0, 0)),
            pl.BlockSpec((n, 1), lambda t, se, cb: (0, 0)),
            pl.BlockSpec((n, 1), lambda t, se, cb: (0, 0)),
        ],
        out_specs=pl.BlockSpec((_K, _RT), lambda t, se, cb: (0, t)),
        scratch_shapes=[pltpu.VMEM((n, _RT), jnp.float32),
                        pltpu.VMEM((n // _CT, _RT), jnp.float32)],
    )
    return pl.pallas_call(
        functools.partial(_knn_body, d=d),
        grid_spec=grid_spec,
        out_shape=jax.ShapeDtypeStruct((_K, n), jnp.int32),
    )(se, cb, x, btr2, sq.reshape(1, n), x, btc2, sq.reshape(n, 1))


def _sc_gather(table, idxflat, d):
    m = idxflat.shape[0]
    info = plsc.get_sparse_core_info()
    nw = info.num_cores * info.num_subcores
    per_w = m // nw
    ch = 128
    nch = per_w // ch
    mesh = plsc.VectorSubcoreMesh(core_axis_name="c", subcore_axis_name="s")

    @functools.partial(
        pl.kernel, mesh=mesh,
        out_type=jax.ShapeDtypeStruct((m, d), jnp.float32),
        scratch_types=[
            pltpu.VMEM((ch,), jnp.int32),
            pltpu.VMEM((ch, d), jnp.float32),
            pltpu.SemaphoreType.DMA,
        ],
    )
    def k(table_hbm, idx_hbm, out_hbm, idx_v, rows_v, sem):
        wid = lax.axis_index("s") * info.num_cores + lax.axis_index("c")
        base = wid * per_w

        def body(q, _):
            off = base + q * ch
            pltpu.sync_copy(idx_hbm.at[pl.ds(off, ch)], idx_v)
            pltpu.async_copy(table_hbm.at[idx_v], rows_v, sem).wait()
            pltpu.sync_copy(rows_v, out_hbm.at[pl.ds(off, ch)])
            return 0

        lax.fori_loop(0, nch, body, 0)

    return k(table, idxflat)


def _conv1_body(g1_ref, cadd_ref, w1_ref, b1_ref, wc2_ref, bc2_ref, wd_ref,
                x1_ref, c2_ref, d2v_ref):
    e = jax.nn.relu(g1_ref[..., :64] + cadd_ref[...][None])
    h = lax.dot_general(e, w1_ref[...], (((2,), (0,)), ((), ())),
                        preferred_element_type=jnp.float32)
    x1 = jnp.max(h, axis=0) + b1_ref[...][None, :]
    x1_ref[...] = x1
    c2_ref[...] = x1 @ wc2_ref[...] + bc2_ref[...][None, :]
    d2v_ref[...] = x1 @ wd_ref[...]


def _aggr_body(x1_ref, c2_ref, g2_ref, batch_ref, wa1_ref, wa2_ref, ab_ref,
               h0w_ref, h0b_ref, h1w_ref, h1b_ref, h2w_ref, h2b_ref,
               out_ref, acc_ref):
    t = pl.program_id(0)
    nsteps = pl.num_programs(0)
    m2 = jnp.max(g2_ref[...], axis=0)
    x2 = c2_ref[...] + m2
    h = (x1_ref[...] @ wa1_ref[...] + x2 @ wa2_ref[...]
         + ab_ref[...][None, :])
    bt = batch_ref[0]
    rows = []
    for b in range(_B):
        rows.append(jnp.max(jnp.where(bt == b, h, -jnp.inf), axis=0,
                            keepdims=True))
    acc = jnp.concatenate(rows, axis=0)

    @pl.when(t == 0)
    def _():
        acc_ref[...] = acc

    @pl.when(t != 0)
    def _():
        acc_ref[...] = jnp.maximum(acc_ref[...], acc)

    @pl.when(t == nsteps - 1)
    def _():
        hh = jax.nn.relu(acc_ref[...] @ h0w_ref[...] + h0b_ref[...][None, :])
        hh = jax.nn.relu(hh @ h1w_ref[...] + h1b_ref[...][None, :])
        out_ref[...] = hh @ h2w_ref[...] + h2b_ref[...][None, :]


def kernel(pos, batch, b0l0_W, b0l0_b, b0l1_W, b0l1_b, b1l0_W, b1l0_b,
           aggr_W, aggr_b, h0_W, h0_b, h1_W, h1_b, h2_W, h2_b):
    batch = batch.astype(jnp.int32)

    arangeb = jnp.arange(_B, dtype=jnp.int32)
    starts = jnp.searchsorted(batch, arangeb, side="left").astype(jnp.int32)
    ends = jnp.searchsorted(batch, arangeb, side="right").astype(jnp.int32)
    se = jnp.stack([starts, ends])
    btile = batch.reshape(_N // _RT, _RT)
    c0 = starts[btile[:, 0]] // _CT
    c1 = (ends[btile[:, -1]] + _CT - 1) // _CT
    cb = jnp.stack([c0, c1], axis=1).astype(jnp.int32)
    btr2 = batch.reshape(1, _N)
    btc2 = batch.reshape(_N, 1)

    pos8 = jnp.pad(pos, ((0, 0), (0, 5)))
    sq1 = jnp.sum(pos * pos, axis=1)
    idx1 = _knn(pos8, btc2, btr2, sq1, se, cb, 8)
    a1 = pos @ b0l0_W[:3] + b0l0_b
    bm1 = pos @ b0l0_W[3:]
    c1_ = a1 - bm1
    bm1p = jnp.pad(bm1, ((0, 0), (0, 64)))
    g1 = _sc_gather(bm1p, idx1.reshape(_K * _N), 128)
    x1, c2, d2v = pl.pallas_call(
        _conv1_body,
        grid=(_N // _RT,),
        in_specs=[
            pl.BlockSpec((_K, _RT, 128), lambda t: (0, t, 0)),
            pl.BlockSpec((_RT, 64), lambda t: (t, 0)),
            pl.BlockSpec((64, 64), lambda t: (0, 0)),
            pl.BlockSpec((64,), lambda t: (0,)),
            pl.BlockSpec((64, 128), lambda t: (0, 0)),
            pl.BlockSpec((128,), lambda t: (0,)),
            pl.BlockSpec((64, 128), lambda t: (0, 0)),
        ],
        out_specs=[
            pl.BlockSpec((_RT, 64), lambda t: (t, 0)),
            pl.BlockSpec((_RT, 128), lambda t: (t, 0)),
            pl.BlockSpec((_RT, 128), lambda t: (t, 0)),
        ],
        out_shape=[
            jax.ShapeDtypeStruct((_N, 64), jnp.float32),
            jax.ShapeDtypeStruct((_N, 128), jnp.float32),
            jax.ShapeDtypeStruct((_N, 128), jnp.float32),
        ],
    )(g1.reshape(_K, _N, 128), c1_, b0l1_W, b0l1_b,
      b1l0_W[:64] - b1l0_W[64:], b1l0_b, b1l0_W[64:])

    sq2 = jnp.sum(x1 * x1, axis=1)
    idx2 = _knn(x1, btc2, btr2, sq2, se, cb, 64)
    g2 = _sc_gather(d2v, idx2.reshape(_K * _N), 128)

    bt3 = batch.reshape(_N // _RT, _RT, 1)
    return pl.pallas_call(
        _aggr_body,
        grid=(_N // _RT,),
        in_specs=[
            pl.BlockSpec((_RT, 64), lambda t: (t, 0)),
            pl.BlockSpec((_RT, 128), lambda t: (t, 0)),
            pl.BlockSpec((_K, _RT, 128), lambda t: (0, t, 0)),
            pl.BlockSpec((1, _RT, 1), lambda t: (t, 0, 0)),
            pl.BlockSpec((64, 1024), lambda t: (0, 0)),
            pl.BlockSpec((128, 1024), lambda t: (0, 0)),
            pl.BlockSpec((1024,), lambda t: (0,)),
            pl.BlockSpec((1024, 512), lambda t: (0, 0)),
            pl.BlockSpec((512,), lambda t: (0,)),
            pl.BlockSpec((512, 256), lambda t: (0, 0)),
            pl.BlockSpec((256,), lambda t: (0,)),
            pl.BlockSpec((256, 40), lambda t: (0, 0)),
            pl.BlockSpec((40,), lambda t: (0,)),
        ],
        out_specs=pl.BlockSpec((_B, 40), lambda t: (0, 0)),
        out_shape=jax.ShapeDtypeStruct((_B, 40), jnp.float32),
        scratch_shapes=[pltpu.VMEM((_B, 1024), jnp.float32)],
    )(x1, c2, g2.reshape(_K, _N, 128), bt3, aggr_W[:64], aggr_W[64:],
      aggr_b, h0_W, h0_b, h1_W, h1_b, h2_W, h2_b)


# speedup vs baseline: 1.8109x; 1.8109x over previous
import functools

import jax
import jax.numpy as jnp
from jax import lax
from jax.experimental import pallas as pl
from jax.experimental.pallas import tpu as pltpu
from jax.experimental.pallas import tpu_sc as plsc

_N = 8192
_B = 8
_K = 20
_RT = 256
_CT = 256


def _knn_body(se_ref, cb_ref, xr_ref, btr_ref, sqr_ref, xc_ref, btc_ref,
              sqc_ref, idx_ref, strip_ref, cmin_ref, *, d):
    t = pl.program_id(0)
    c0 = cb_ref[t, 0]
    c1 = cb_ref[t, 1]
    nt = strip_ref.shape[0] // _CT

    rr = xr_ref[...]
    sqr = sqr_ref[...]
    btr = btr_ref[...]

    inf = jnp.float32(jnp.inf)
    big = jnp.int32(2 * _N)
    cmin_ref[...] = jnp.full(cmin_ref.shape, inf)

    def dist_tile(c, _):
        cc = xc_ref[pl.ds(c * _CT, _CT), :]
        g = jax.lax.dot_general(cc, rr, (((1,), (1,)), ((), ())),
                                preferred_element_type=jnp.float32)
        sqc = sqc_ref[pl.ds(c * _CT, _CT), :]
        d2 = (sqc + sqr) - 2.0 * g
        btc = btc_ref[pl.ds(c * _CT, _CT), :]
        d2 = jnp.where(btc != btr, inf, d2)
        strip_ref[pl.ds(c * _CT, _CT), :] = d2
        cmin_ref[pl.ds(c, 1), :] = jnp.min(d2, axis=0, keepdims=True)
        return 0

    jax.lax.fori_loop(c0, c1, dist_tile, 0)

    s_row = jnp.zeros((1, _RT), jnp.int32)
    e_row = jnp.zeros((1, _RT), jnp.int32)
    for b in range(_B):
        s_row = jnp.where(btr == b, se_ref[0, b], s_row)
        e_row = jnp.where(btr == b, se_ref[1, b], e_row)
    nvalid = e_row - s_row

    iota0 = jax.lax.broadcasted_iota(jnp.int32, (_CT, _RT), 0)

    riota = jax.lax.broadcasted_iota(jnp.int32, (nt, _RT), 0)

    for k in range(_K):
        cm = cmin_ref[...]
        m = jnp.min(cm, axis=0, keepdims=True)
        tc = jnp.min(jnp.where(cm == m, riota, big), axis=0, keepdims=True)

        def pickpass(c, idx):
            rowsel = tc == c
            tile = strip_ref[pl.ds(c * _CT, _CT), :]
            eq = (tile == m) & rowsel
            jstar = jnp.min(jnp.where(eq, iota0, big), axis=0, keepdims=True)
            newtile = jnp.where(iota0 == jstar, inf, tile)
            strip_ref[pl.ds(c * _CT, _CT), :] = newtile
            cmin_ref[pl.ds(c, 1), :] = jnp.min(newtile, axis=0, keepdims=True)
            return jnp.where(rowsel, c * _CT + jstar, idx)

        idx = jax.lax.fori_loop(c0, c1, pickpass, jnp.full((1, _RT), big))

        p = k - nvalid
        idxfix = jnp.where(p < s_row, p, e_row + (p - s_row))
        idx = jnp.where(m == inf, idxfix, idx)
        idx_ref[k:k + 1, :] = idx


def _knn(x, btc2, btr2, sq, se, cb, d):
    n = x.shape[0]
    grid_spec = pltpu.PrefetchScalarGridSpec(
        num_scalar_prefetch=2,
        grid=(n // _RT,),
        in_specs=[
            pl.BlockSpec((_RT, d), lambda t, se, cb: (t, 0)),
            pl.BlockSpec((1, _RT), lambda t, se, cb: (0, t)),
            pl.BlockSpec((1, _RT), lambda t, se, cb: (0, t)),
            pl.BlockSpec((n, d), lambda t, se, cb: (0, 0)),
            pl.BlockSpec((n, 1), lambda t, se, cb: (0, 0)),
            pl.BlockSpec((n, 1), lambda t, se, cb: (0, 0)),
        ],
        out_specs=pl.BlockSpec((_K, _RT), lambda t, se, cb: (0, t)),
        scratch_shapes=[pltpu.VMEM((n, _RT), jnp.float32),
                        pltpu.VMEM((n // _CT, _RT), jnp.float32)],
    )
    return pl.pallas_call(
        functools.partial(_knn_body, d=d),
        grid_spec=grid_spec,
        out_shape=jax.ShapeDtypeStruct((_K, n), jnp.int32),
    )(se, cb, x, btr2, sq.reshape(1, n), x, btc2, sq.reshape(n, 1))


def _sc_gather(table, idxflat, d):
    m = idxflat.shape[0]
    info = plsc.get_sparse_core_info()
    nw = info.num_cores * info.num_subcores
    per_w = m // nw
    ch = 128
    nch = per_w // ch
    mesh = plsc.VectorSubcoreMesh(core_axis_name="c", subcore_axis_name="s")

    @functools.partial(
        pl.kernel, mesh=mesh,
        out_type=jax.ShapeDtypeStruct((m, d), jnp.float32),
        scratch_types=[
            pltpu.VMEM((ch,), jnp.int32),
            pltpu.VMEM((ch, d), jnp.float32),
            pltpu.SemaphoreType.DMA,
        ],
    )
    def k(table_hbm, idx_hbm, out_hbm, idx_v, rows_v, sem):
        wid = lax.axis_index("s") * info.num_cores + lax.axis_index("c")
        base = wid * per_w

        def body(q, _):
            off = base + q * ch
            pltpu.sync_copy(idx_hbm.at[pl.ds(off, ch)], idx_v)
            pltpu.async_copy(table_hbm.at[idx_v], rows_v, sem).wait()
            pltpu.sync_copy(rows_v, out_hbm.at[pl.ds(off, ch)])
            return 0

        lax.fori_loop(0, nch, body, 0)

    return k(table, idxflat)


def _conv1_body(g1_ref, cadd_ref, w1_ref, b1_ref, wc2_ref, bc2_ref, wd_ref,
                x1_ref, c2_ref, d2v_ref):
    e = jax.nn.relu(g1_ref[..., :64] + cadd_ref[...][None])
    h = lax.dot_general(e, w1_ref[...], (((2,), (0,)), ((), ())),
                        preferred_element_type=jnp.float32)
    x1 = jnp.max(h, axis=0) + b1_ref[...][None, :]
    x1_ref[...] = x1
    c2_ref[...] = x1 @ wc2_ref[...] + bc2_ref[...][None, :]
    d2v_ref[...] = x1 @ wd_ref[...]


def _aggr_body(x1_ref, c2_ref, g2_ref, batch_ref, wa1_ref, wa2_ref, ab_ref,
               h0w_ref, h0b_ref, h1w_ref, h1b_ref, h2w_ref, h2b_ref,
               out_ref, acc_ref):
    t = pl.program_id(0)
    nsteps = pl.num_programs(0)
    m2 = jnp.max(g2_ref[...], axis=0)
    x2 = c2_ref[...] + m2
    h = (x1_ref[...] @ wa1_ref[...] + x2 @ wa2_ref[...]
         + ab_ref[...][None, :])
    bt = batch_ref[0]
    rows = []
    for b in range(_B):
        rows.append(jnp.max(jnp.where(bt == b, h, -jnp.inf), axis=0,
                            keepdims=True))
    acc = jnp.concatenate(rows, axis=0)

    @pl.when(t == 0)
    def _():
        acc_ref[...] = acc

    @pl.when(t != 0)
    def _():
        acc_ref[...] = jnp.maximum(acc_ref[...], acc)

    @pl.when(t == nsteps - 1)
    def _():
        hh = jax.nn.relu(acc_ref[...] @ h0w_ref[...] + h0b_ref[...][None, :])
        hh = jax.nn.relu(hh @ h1w_ref[...] + h1b_ref[...][None, :])
        out_ref[...] = hh @ h2w_ref[...] + h2b_ref[...][None, :]


def kernel(pos, batch, b0l0_W, b0l0_b, b0l1_W, b0l1_b, b1l0_W, b1l0_b,
           aggr_W, aggr_b, h0_W, h0_b, h1_W, h1_b, h2_W, h2_b):
    batch = batch.astype(jnp.int32)

    arangeb = jnp.arange(_B, dtype=jnp.int32)
    starts = jnp.searchsorted(batch, arangeb, side="left").astype(jnp.int32)
    ends = jnp.searchsorted(batch, arangeb, side="right").astype(jnp.int32)
    se = jnp.stack([starts, ends])
    btile = batch.reshape(_N // _RT, _RT)
    c0 = starts[btile[:, 0]] // _CT
    c1 = (ends[btile[:, -1]] + _CT - 1) // _CT
    cb = jnp.stack([c0, c1], axis=1).astype(jnp.int32)
    btr2 = batch.reshape(1, _N)
    btc2 = batch.reshape(_N, 1)

    pos8 = jnp.pad(pos, ((0, 0), (0, 5)))
    sq1 = jnp.sum(pos * pos, axis=1)
    idx1 = _knn(pos8, btc2, btr2, sq1, se, cb, 8)
    a1 = pos @ b0l0_W[:3] + b0l0_b
    bm1 = pos @ b0l0_W[3:]
    c1_ = a1 - bm1
    bm1p = jnp.pad(bm1, ((0, 0), (0, 64)))
    g1 = _sc_gather(bm1p, idx1.reshape(_K * _N), 128)
    x1, c2, d2v = pl.pallas_call(
        _conv1_body,
        grid=(_N // _RT,),
        in_specs=[
            pl.BlockSpec((_K, _RT, 128), lambda t: (0, t, 0)),
            pl.BlockSpec((_RT, 64), lambda t: (t, 0)),
            pl.BlockSpec((64, 64), lambda t: (0, 0)),
            pl.BlockSpec((64,), lambda t: (0,)),
            pl.BlockSpec((64, 128), lambda t: (0, 0)),
            pl.BlockSpec((128,), lambda t: (0,)),
            pl.BlockSpec((64, 128), lambda t: (0, 0)),
        ],
        out_specs=[
            pl.BlockSpec((_RT, 64), lambda t: (t, 0)),
            pl.BlockSpec((_RT, 128), lambda t: (t, 0)),
            pl.BlockSpec((_RT, 128), lambda t: (t, 0)),
        ],
        out_shape=[
            jax.ShapeDtypeStruct((_N, 64), jnp.float32),
            jax.ShapeDtypeStruct((_N, 128), jnp.float32),
            jax.ShapeDtypeStruct((_N, 128), jnp.float32),
        ],
    )(g1.reshape(_K, _N, 128), c1_, b0l1_W, b0l1_b,
      b1l0_W[:64] - b1l0_W[64:], b1l0_b, b1l0_W[64:])

    sq2 = jnp.sum(x1 * x1, axis=1)
    idx2 = _knn(x1, btc2, btr2, sq2, se, cb, 64)
    g2 = _sc_gather(d2v, idx2.reshape(_K * _N), 128)

    bt3 = batch.reshape(_N // _RT, _RT, 1)
    return pl.pallas_call(
        _aggr_body,
        grid=(_N // _RT,),
        in_specs=[
            pl.BlockSpec((_RT, 64), lambda t: (t, 0)),
            pl.BlockSpec((_RT, 128), lambda t: (t, 0)),
            pl.BlockSpec((_K, _RT, 128), lambda t: (0, t, 0)),
            pl.BlockSpec((1, _RT, 1), lambda t: (t, 0, 0)),
            pl.BlockSpec((64, 1024), lambda t: (0, 0)),
            pl.BlockSpec((128, 1024), lambda t: (0, 0)),
            pl.BlockSpec((1024,), lambda t: (0,)),
            pl.BlockSpec((1024, 512), lambda t: (0, 0)),
            pl.BlockSpec((512,), lambda t: (0,)),
            pl.BlockSpec((512, 256), lambda t: (0, 0)),
            pl.BlockSpec((256,), lambda t: (0,)),
            pl.BlockSpec((256, 40), lambda t: (0, 0)),
            pl.BlockSpec((40,), lambda t: (0,)),
        ],
        out_specs=pl.BlockSpec((_B, 40), lambda t: (0, 0)),
        out_shape=jax.ShapeDtypeStruct((_B, 40), jnp.float32),
        scratch_shapes=[pltpu.VMEM((_B, 1024), jnp.float32)],
    )(x1, c2, g2.reshape(_K, _N, 128), bt3, aggr_W[:64], aggr_W[64:],
      aggr_b, h0_W, h0_b, h1_W, h1_b, h2_W, h2_b)


# speedup vs baseline: 1.8203x; 1.0051x over previous
import functools

import jax
import jax.numpy as jnp
from jax import lax
from jax.experimental import pallas as pl
from jax.experimental.pallas import tpu as pltpu
from jax.experimental.pallas import tpu_sc as plsc

_N = 8192
_B = 8
_K = 20
_RT = 256
_CT = 256


def _knn_body(se_ref, cb_ref, xr_ref, btr_ref, sqr_ref, xc_ref, btc_ref,
              sqc_ref, idx_ref, strip_ref, cmin_ref, *, d):
    t = pl.program_id(0)
    c0 = cb_ref[t, 0]
    c1 = cb_ref[t, 1]
    nt = strip_ref.shape[0] // _CT

    rr = xr_ref[...]
    sqr = sqr_ref[...]
    btr = btr_ref[...]

    inf = jnp.float32(jnp.inf)
    big = jnp.int32(2 * _N)
    cmin_ref[...] = jnp.full(cmin_ref.shape, inf)

    def dist_tile(c, _):
        cc = xc_ref[pl.ds(c * _CT, _CT), :]
        g = jax.lax.dot_general(cc, rr, (((1,), (1,)), ((), ())),
                                preferred_element_type=jnp.float32)
        sqc = sqc_ref[pl.ds(c * _CT, _CT), :]
        d2 = (sqc + sqr) - 2.0 * g
        btc = btc_ref[pl.ds(c * _CT, _CT), :]
        d2 = jnp.where(btc != btr, inf, d2)
        strip_ref[pl.ds(c * _CT, _CT), :] = d2
        cmin_ref[pl.ds(c, 1), :] = jnp.min(d2, axis=0, keepdims=True)
        return 0

    jax.lax.fori_loop(c0, c1, dist_tile, 0)

    s_row = jnp.zeros((1, _RT), jnp.int32)
    e_row = jnp.zeros((1, _RT), jnp.int32)
    for b in range(_B):
        s_row = jnp.where(btr == b, se_ref[0, b], s_row)
        e_row = jnp.where(btr == b, se_ref[1, b], e_row)
    nvalid = e_row - s_row

    iota0 = jax.lax.broadcasted_iota(jnp.int32, (_CT, _RT), 0)

    riota = jax.lax.broadcasted_iota(jnp.int32, (nt, _RT), 0)

    for k in range(_K):
        cm = cmin_ref[...]
        m = jnp.min(cm, axis=0, keepdims=True)
        tc = jnp.min(jnp.where(cm == m, riota, big), axis=0, keepdims=True)

        def pickpass(c, idx):
            rowsel = tc == c
            tile = strip_ref[pl.ds(c * _CT, _CT), :]
            eq = (tile == m) & rowsel
            jstar = jnp.min(jnp.where(eq, iota0, big), axis=0, keepdims=True)
            newtile = jnp.where(iota0 == jstar, inf, tile)
            strip_ref[pl.ds(c * _CT, _CT), :] = newtile
            cmin_ref[pl.ds(c, 1), :] = jnp.min(newtile, axis=0, keepdims=True)
            return jnp.where(rowsel, c * _CT + jstar, idx)

        idx = jax.lax.fori_loop(c0, c1, pickpass, jnp.full((1, _RT), big))

        p = k - nvalid
        idxfix = jnp.where(p < s_row, p, e_row + (p - s_row))
        idx = jnp.where(m == inf, idxfix, idx)
        idx_ref[k:k + 1, :] = idx


def _knn(x, btc2, btr2, sq, se, cb, d):
    n = x.shape[0]
    grid_spec = pltpu.PrefetchScalarGridSpec(
        num_scalar_prefetch=2,
        grid=(n // _RT,),
        in_specs=[
            pl.BlockSpec((_RT, d), lambda t, se, cb: (t, 0)),
            pl.BlockSpec((1, _RT), lambda t, se, cb: (0, t)),
            pl.BlockSpec((1, _RT), lambda t, se, cb: (0, t)),
            pl.BlockSpec((n, d), lambda t, se, cb: (0, 0)),
            pl.BlockSpec((n, 1), lambda t, se, cb: (0, 0)),
            pl.BlockSpec((n, 1), lambda t, se, cb: (0, 0)),
        ],
        out_specs=pl.BlockSpec((_K, _RT), lambda t, se, cb: (0, t)),
        scratch_shapes=[pltpu.VMEM((n, _RT), jnp.float32),
                        pltpu.VMEM((n // _CT, _RT), jnp.float32)],
    )
    return pl.pallas_call(
        functools.partial(_knn_body, d=d),
        grid_spec=grid_spec,
        out_shape=jax.ShapeDtypeStruct((_K, n), jnp.int32),
    )(se, cb, x, btr2, sq.reshape(1, n), x, btc2, sq.reshape(n, 1))


def _sc_gather(table, idxflat, d):
    m = idxflat.shape[0]
    info = plsc.get_sparse_core_info()
    nw = info.num_cores * info.num_subcores
    per_w = m // nw
    ch = 128
    nch = per_w // ch
    mesh = plsc.VectorSubcoreMesh(core_axis_name="c", subcore_axis_name="s")

    @functools.partial(
        pl.kernel, mesh=mesh,
        out_type=jax.ShapeDtypeStruct((m, d), jnp.float32),
        scratch_types=[
            pltpu.VMEM((ch,), jnp.int32),
            pltpu.VMEM((ch, d), jnp.float32),
            pltpu.SemaphoreType.DMA,
        ],
    )
    def k(table_hbm, idx_hbm, out_hbm, idx_v, rows_v, sem):
        wid = lax.axis_index("s") * info.num_cores + lax.axis_index("c")
        base = wid * per_w

        def body(q, _):
            off = base + q * ch
            pltpu.sync_copy(idx_hbm.at[pl.ds(off, ch)], idx_v)
            pltpu.async_copy(table_hbm.at[idx_v], rows_v, sem).wait()
            pltpu.sync_copy(rows_v, out_hbm.at[pl.ds(off, ch)])
            return 0

        lax.fori_loop(0, nch, body, 0)

    return k(table, idxflat)


def _sc_gather_max(table, idx, d):
    n = table.shape[0]
    ktot = idx.shape[0]
    info = plsc.get_sparse_core_info()
    nw = info.num_cores * info.num_subcores
    per_w = n // nw
    ch = 128
    nch = per_w // ch
    mesh = plsc.VectorSubcoreMesh(core_axis_name="c", subcore_axis_name="s")

    @functools.partial(
        pl.kernel, mesh=mesh,
        out_type=jax.ShapeDtypeStruct((n, d), jnp.float32),
        scratch_types=[
            pltpu.VMEM((ch,), jnp.int32),
            pltpu.VMEM((ch, d), jnp.float32),
            pltpu.VMEM((ch, d), jnp.float32),
            pltpu.SemaphoreType.DMA,
        ],
    )
    def k(table_hbm, idx_hbm, out_hbm, idx_v, rows_v, acc_v, sem):
        wid = lax.axis_index("s") * info.num_cores + lax.axis_index("c")

        def chunk(q, _):
            pbase = wid * per_w + q * ch
            pltpu.sync_copy(idx_hbm.at[0, pl.ds(pbase, ch)], idx_v)
            pltpu.async_copy(table_hbm.at[idx_v], acc_v, sem).wait()

            def kstep(kk, _):
                pltpu.sync_copy(idx_hbm.at[kk, pl.ds(pbase, ch)], idx_v)
                pltpu.async_copy(table_hbm.at[idx_v], rows_v, sem).wait()

                def rowloop(r4, _):
                    for u in range(4):
                        r = r4 * 4 + u
                        for j in range(d // 16):
                            sl = pl.ds(j * 16, 16)
                            acc_v[r, sl] = jnp.maximum(acc_v[r, sl],
                                                       rows_v[r, sl])
                    return 0

                lax.fori_loop(0, ch // 4, rowloop, 0)
                return 0

            lax.fori_loop(1, ktot, kstep, 0)
            pltpu.sync_copy(acc_v, out_hbm.at[pl.ds(pbase, ch)])
            return 0

        lax.fori_loop(0, nch, chunk, 0)

    return k(table, idx)


def _conv1_body(g1_ref, cadd_ref, w1_ref, b1_ref, wc2_ref, bc2_ref, wd_ref,
                x1_ref, c2_ref, d2v_ref):
    e = jax.nn.relu(g1_ref[..., :64] + cadd_ref[...][None])
    h = lax.dot_general(e, w1_ref[...], (((2,), (0,)), ((), ())),
                        preferred_element_type=jnp.float32)
    x1 = jnp.max(h, axis=0) + b1_ref[...][None, :]
    x1_ref[...] = x1
    c2_ref[...] = x1 @ wc2_ref[...] + bc2_ref[...][None, :]
    d2v_ref[...] = x1 @ wd_ref[...]


def _aggr_body(x1_ref, c2_ref, m2_ref, batch_ref, wa1_ref, wa2_ref, ab_ref,
               h0w_ref, h0b_ref, h1w_ref, h1b_ref, h2w_ref, h2b_ref,
               out_ref, acc_ref):
    t = pl.program_id(0)
    nsteps = pl.num_programs(0)
    x2 = c2_ref[...] + m2_ref[...]
    h = (x1_ref[...] @ wa1_ref[...] + x2 @ wa2_ref[...]
         + ab_ref[...][None, :])
    bt = batch_ref[0]
    rows = []
    for b in range(_B):
        rows.append(jnp.max(jnp.where(bt == b, h, -jnp.inf), axis=0,
                            keepdims=True))
    acc = jnp.concatenate(rows, axis=0)

    @pl.when(t == 0)
    def _():
        acc_ref[...] = acc

    @pl.when(t != 0)
    def _():
        acc_ref[...] = jnp.maximum(acc_ref[...], acc)

    @pl.when(t == nsteps - 1)
    def _():
        hh = jax.nn.relu(acc_ref[...] @ h0w_ref[...] + h0b_ref[...][None, :])
        hh = jax.nn.relu(hh @ h1w_ref[...] + h1b_ref[...][None, :])
        out_ref[...] = hh @ h2w_ref[...] + h2b_ref[...][None, :]


def kernel(pos, batch, b0l0_W, b0l0_b, b0l1_W, b0l1_b, b1l0_W, b1l0_b,
           aggr_W, aggr_b, h0_W, h0_b, h1_W, h1_b, h2_W, h2_b):
    batch = batch.astype(jnp.int32)

    arangeb = jnp.arange(_B, dtype=jnp.int32)
    starts = jnp.searchsorted(batch, arangeb, side="left").astype(jnp.int32)
    ends = jnp.searchsorted(batch, arangeb, side="right").astype(jnp.int32)
    se = jnp.stack([starts, ends])
    btile = batch.reshape(_N // _RT, _RT)
    c0 = starts[btile[:, 0]] // _CT
    c1 = (ends[btile[:, -1]] + _CT - 1) // _CT
    cb = jnp.stack([c0, c1], axis=1).astype(jnp.int32)
    btr2 = batch.reshape(1, _N)
    btc2 = batch.reshape(_N, 1)

    pos8 = jnp.pad(pos, ((0, 0), (0, 5)))
    sq1 = jnp.sum(pos * pos, axis=1)
    idx1 = _knn(pos8, btc2, btr2, sq1, se, cb, 8)
    a1 = pos @ b0l0_W[:3] + b0l0_b
    bm1 = pos @ b0l0_W[3:]
    c1_ = a1 - bm1
    bm1p = jnp.pad(bm1, ((0, 0), (0, 64)))
    g1 = _sc_gather(bm1p, idx1.reshape(_K * _N), 128)
    x1, c2, d2v = pl.pallas_call(
        _conv1_body,
        grid=(_N // _RT,),
        in_specs=[
            pl.BlockSpec((_K, _RT, 128), lambda t: (0, t, 0)),
            pl.BlockSpec((_RT, 64), lambda t: (t, 0)),
            pl.BlockSpec((64, 64), lambda t: (0, 0)),
            pl.BlockSpec((64,), lambda t: (0,)),
            pl.BlockSpec((64, 128), lambda t: (0, 0)),
            pl.BlockSpec((128,), lambda t: (0,)),
            pl.BlockSpec((64, 128), lambda t: (0, 0)),
        ],
        out_specs=[
            pl.BlockSpec((_RT, 64), lambda t: (t, 0)),
            pl.BlockSpec((_RT, 128), lambda t: (t, 0)),
            pl.BlockSpec((_RT, 128), lambda t: (t, 0)),
        ],
        out_shape=[
            jax.ShapeDtypeStruct((_N, 64), jnp.float32),
            jax.ShapeDtypeStruct((_N, 128), jnp.float32),
            jax.ShapeDtypeStruct((_N, 128), jnp.float32),
        ],
    )(g1.reshape(_K, _N, 128), c1_, b0l1_W, b0l1_b,
      b1l0_W[:64] - b1l0_W[64:], b1l0_b, b1l0_W[64:])

    sq2 = jnp.sum(x1 * x1, axis=1)
    idx2 = _knn(x1, btc2, btr2, sq2, se, cb, 64)
    m2 = _sc_gather_max(d2v, idx2, 128)

    bt3 = batch.reshape(_N // _RT, _RT, 1)
    return pl.pallas_call(
        _aggr_body,
        grid=(_N // _RT,),
        in_specs=[
            pl.BlockSpec((_RT, 64), lambda t: (t, 0)),
            pl.BlockSpec((_RT, 128), lambda t: (t, 0)),
            pl.BlockSpec((_RT, 128), lambda t: (t, 0)),
            pl.BlockSpec((1, _RT, 1), lambda t: (t, 0, 0)),
            pl.BlockSpec((64, 1024), lambda t: (0, 0)),
            pl.BlockSpec((128, 1024), lambda t: (0, 0)),
            pl.BlockSpec((1024,), lambda t: (0,)),
            pl.BlockSpec((1024, 512), lambda t: (0, 0)),
            pl.BlockSpec((512,), lambda t: (0,)),
            pl.BlockSpec((512, 256), lambda t: (0, 0)),
            pl.BlockSpec((256,), lambda t: (0,)),
            pl.BlockSpec((256, 40), lambda t: (0, 0)),
            pl.BlockSpec((40,), lambda t: (0,)),
        ],
        out_specs=pl.BlockSpec((_B, 40), lambda t: (0, 0)),
        out_shape=jax.ShapeDtypeStruct((_B, 40), jnp.float32),
        scratch_shapes=[pltpu.VMEM((_B, 1024), jnp.float32)],
    )(x1, c2, m2, bt3, aggr_W[:64], aggr_W[64:],
      aggr_b, h0_W, h0_b, h1_W, h1_b, h2_W, h2_b)


# speedup vs baseline: 1.8206x; 1.0002x over previous
import functools

import jax
import jax.numpy as jnp
from jax import lax
from jax.experimental import pallas as pl
from jax.experimental.pallas import tpu as pltpu
from jax.experimental.pallas import tpu_sc as plsc

_N = 8192
_B = 8
_K = 20
_RT = 256
_CT = 256


def _knn_body(se_ref, cb_ref, xr_ref, btr_ref, sqr_ref, xc_ref, btc_ref,
              sqc_ref, idx_ref, strip_ref, cmin_ref, *, d):
    t = pl.program_id(0)
    c0 = cb_ref[t, 0]
    c1 = cb_ref[t, 1]
    nt = strip_ref.shape[0] // _CT

    rr = xr_ref[...]
    sqr = sqr_ref[...]
    btr = btr_ref[...]

    inf = jnp.float32(jnp.inf)
    big = jnp.int32(2 * _N)
    cmin_ref[...] = jnp.full(cmin_ref.shape, inf)

    def dist_tile(c, _):
        cc = xc_ref[pl.ds(c * _CT, _CT), :]
        g = jax.lax.dot_general(cc, rr, (((1,), (1,)), ((), ())),
                                preferred_element_type=jnp.float32)
        sqc = sqc_ref[pl.ds(c * _CT, _CT), :]
        d2 = (sqc + sqr) - 2.0 * g
        btc = btc_ref[pl.ds(c * _CT, _CT), :]
        d2 = jnp.where(btc != btr, inf, d2)
        strip_ref[pl.ds(c * _CT, _CT), :] = d2
        cmin_ref[pl.ds(c, 1), :] = jnp.min(d2, axis=0, keepdims=True)
        return 0

    jax.lax.fori_loop(c0, c1, dist_tile, 0)

    s_row = jnp.zeros((1, _RT), jnp.int32)
    e_row = jnp.zeros((1, _RT), jnp.int32)
    for b in range(_B):
        s_row = jnp.where(btr == b, se_ref[0, b], s_row)
        e_row = jnp.where(btr == b, se_ref[1, b], e_row)
    nvalid = e_row - s_row

    iota0 = jax.lax.broadcasted_iota(jnp.int32, (_CT, _RT), 0)

    riota = jax.lax.broadcasted_iota(jnp.int32, (nt, _RT), 0)

    for k in range(_K):
        cm = cmin_ref[...]
        m = jnp.min(cm, axis=0, keepdims=True)
        tc = jnp.min(jnp.where(cm == m, riota, big), axis=0, keepdims=True)

        def pickpass(c, idx):
            rowsel = tc == c
            tile = strip_ref[pl.ds(c * _CT, _CT), :]
            eq = (tile == m) & rowsel
            jstar = jnp.min(jnp.where(eq, iota0, big), axis=0, keepdims=True)
            newtile = jnp.where(iota0 == jstar, inf, tile)
            strip_ref[pl.ds(c * _CT, _CT), :] = newtile
            cmin_ref[pl.ds(c, 1), :] = jnp.min(newtile, axis=0, keepdims=True)
            return jnp.where(rowsel, c * _CT + jstar, idx)

        idx = jax.lax.fori_loop(c0, c1, pickpass, jnp.full((1, _RT), big))

        p = k - nvalid
        idxfix = jnp.where(p < s_row, p, e_row + (p - s_row))
        idx = jnp.where(m == inf, idxfix, idx)
        idx_ref[k:k + 1, :] = idx


def _knn(x, btc2, btr2, sq, se, cb, d):
    n = x.shape[0]
    grid_spec = pltpu.PrefetchScalarGridSpec(
        num_scalar_prefetch=2,
        grid=(n // _RT,),
        in_specs=[
            pl.BlockSpec((_RT, d), lambda t, se, cb: (t, 0)),
            pl.BlockSpec((1, _RT), lambda t, se, cb: (0, t)),
            pl.BlockSpec((1, _RT), lambda t, se, cb: (0, t)),
            pl.BlockSpec((n, d), lambda t, se, cb: (0, 0)),
            pl.BlockSpec((n, 1), lambda t, se, cb: (0, 0)),
            pl.BlockSpec((n, 1), lambda t, se, cb: (0, 0)),
        ],
        out_specs=pl.BlockSpec((_K, _RT), lambda t, se, cb: (0, t)),
        scratch_shapes=[pltpu.VMEM((n, _RT), jnp.float32),
                        pltpu.VMEM((n // _CT, _RT), jnp.float32)],
    )
    return pl.pallas_call(
        functools.partial(_knn_body, d=d),
        grid_spec=grid_spec,
        out_shape=jax.ShapeDtypeStruct((_K, n), jnp.int32),
    )(se, cb, x, btr2, sq.reshape(1, n), x, btc2, sq.reshape(n, 1))


def _sc_gather(table, idxflat, d, dout=None):
    m = idxflat.shape[0]
    dout = d if dout is None else dout
    info = plsc.get_sparse_core_info()
    nw = info.num_cores * info.num_subcores
    per_w = m // nw
    ch = 128
    nch = per_w // ch
    mesh = plsc.VectorSubcoreMesh(core_axis_name="c", subcore_axis_name="s")

    @functools.partial(
        pl.kernel, mesh=mesh,
        out_type=jax.ShapeDtypeStruct((m, dout), jnp.float32),
        scratch_types=[
            pltpu.VMEM((ch,), jnp.int32),
            pltpu.VMEM((ch, d), jnp.float32),
            pltpu.SemaphoreType.DMA,
        ],
    )
    def k(table_hbm, idx_hbm, out_hbm, idx_v, rows_v, sem):
        wid = lax.axis_index("s") * info.num_cores + lax.axis_index("c")
        base = wid * per_w

        def body(q, _):
            off = base + q * ch
            pltpu.sync_copy(idx_hbm.at[pl.ds(off, ch)], idx_v)
            pltpu.async_copy(table_hbm.at[idx_v], rows_v, sem).wait()
            if dout == d:
                pltpu.sync_copy(rows_v, out_hbm.at[pl.ds(off, ch)])
            else:
                pltpu.sync_copy(rows_v.at[:, pl.ds(0, dout)],
                                out_hbm.at[pl.ds(off, ch)])
            return 0

        lax.fori_loop(0, nch, body, 0)

    return k(table, idxflat)


def _sc_gather_max(table, idx, d):
    n = table.shape[0]
    ktot = idx.shape[0]
    info = plsc.get_sparse_core_info()
    nw = info.num_cores * info.num_subcores
    per_w = n // nw
    ch = 128
    nch = per_w // ch
    mesh = plsc.VectorSubcoreMesh(core_axis_name="c", subcore_axis_name="s")

    @functools.partial(
        pl.kernel, mesh=mesh,
        out_type=jax.ShapeDtypeStruct((n, d), jnp.float32),
        scratch_types=[
            pltpu.VMEM((ch,), jnp.int32),
            pltpu.VMEM((ch, d), jnp.float32),
            pltpu.VMEM((ch, d), jnp.float32),
            pltpu.SemaphoreType.DMA,
        ],
    )
    def k(table_hbm, idx_hbm, out_hbm, idx_v, rows_v, acc_v, sem):
        wid = lax.axis_index("s") * info.num_cores + lax.axis_index("c")

        def chunk(q, _):
            pbase = wid * per_w + q * ch
            pltpu.sync_copy(idx_hbm.at[0, pl.ds(pbase, ch)], idx_v)
            pltpu.async_copy(table_hbm.at[idx_v], acc_v, sem).wait()

            def kstep(kk, _):
                pltpu.sync_copy(idx_hbm.at[kk, pl.ds(pbase, ch)], idx_v)
                pltpu.async_copy(table_hbm.at[idx_v], rows_v, sem).wait()

                def rowloop(r4, _):
                    for u in range(4):
                        r = r4 * 4 + u
                        for j in range(d // 16):
                            sl = pl.ds(j * 16, 16)
                            acc_v[r, sl] = jnp.maximum(acc_v[r, sl],
                                                       rows_v[r, sl])
                    return 0

                lax.fori_loop(0, ch // 4, rowloop, 0)
                return 0

            lax.fori_loop(1, ktot, kstep, 0)
            pltpu.sync_copy(acc_v, out_hbm.at[pl.ds(pbase, ch)])
            return 0

        lax.fori_loop(0, nch, chunk, 0)

    return k(table, idx)


def _conv1_body(g1_ref, cadd_ref, w1_ref, b1_ref, wc2_ref, bc2_ref, wd_ref,
                x1_ref, c2_ref, d2v_ref):
    e = jax.nn.relu(g1_ref[..., :64] + cadd_ref[...][None])
    h = lax.dot_general(e, w1_ref[...], (((2,), (0,)), ((), ())),
                        preferred_element_type=jnp.float32)
    x1 = jnp.max(h, axis=0) + b1_ref[...][None, :]
    x1_ref[...] = x1
    c2_ref[...] = x1 @ wc2_ref[...] + bc2_ref[...][None, :]
    d2v_ref[...] = x1 @ wd_ref[...]


def _aggr_body(x1_ref, c2_ref, m2_ref, batch_ref, wa1_ref, wa2_ref, ab_ref,
               h0w_ref, h0b_ref, h1w_ref, h1b_ref, h2w_ref, h2b_ref,
               out_ref, acc_ref):
    t = pl.program_id(0)
    nsteps = pl.num_programs(0)
    x2 = c2_ref[...] + m2_ref[...]
    h = (x1_ref[...] @ wa1_ref[...] + x2 @ wa2_ref[...]
         + ab_ref[...][None, :])
    bt = batch_ref[0]
    rows = []
    for b in range(_B):
        rows.append(jnp.max(jnp.where(bt == b, h, -jnp.inf), axis=0,
                            keepdims=True))
    acc = jnp.concatenate(rows, axis=0)

    @pl.when(t == 0)
    def _():
        acc_ref[...] = acc

    @pl.when(t != 0)
    def _():
        acc_ref[...] = jnp.maximum(acc_ref[...], acc)

    @pl.when(t == nsteps - 1)
    def _():
        hh = jax.nn.relu(acc_ref[...] @ h0w_ref[...] + h0b_ref[...][None, :])
        hh = jax.nn.relu(hh @ h1w_ref[...] + h1b_ref[...][None, :])
        out_ref[...] = hh @ h2w_ref[...] + h2b_ref[...][None, :]


def kernel(pos, batch, b0l0_W, b0l0_b, b0l1_W, b0l1_b, b1l0_W, b1l0_b,
           aggr_W, aggr_b, h0_W, h0_b, h1_W, h1_b, h2_W, h2_b):
    batch = batch.astype(jnp.int32)

    arangeb = jnp.arange(_B, dtype=jnp.int32)
    starts = jnp.searchsorted(batch, arangeb, side="left").astype(jnp.int32)
    ends = jnp.searchsorted(batch, arangeb, side="right").astype(jnp.int32)
    se = jnp.stack([starts, ends])
    btile = batch.reshape(_N // _RT, _RT)
    c0 = starts[btile[:, 0]] // _CT
    c1 = (ends[btile[:, -1]] + _CT - 1) // _CT
    cb = jnp.stack([c0, c1], axis=1).astype(jnp.int32)
    btr2 = batch.reshape(1, _N)
    btc2 = batch.reshape(_N, 1)

    pos8 = jnp.pad(pos, ((0, 0), (0, 5)))
    sq1 = jnp.sum(pos * pos, axis=1)
    idx1 = _knn(pos8, btc2, btr2, sq1, se, cb, 8)
    a1 = pos @ b0l0_W[:3] + b0l0_b
    bm1 = pos @ b0l0_W[3:]
    c1_ = a1 - bm1
    bm1p = jnp.pad(bm1, ((0, 0), (0, 64)))
    g1 = _sc_gather(bm1p, idx1.reshape(_K * _N), 128)
    x1, c2, d2v = pl.pallas_call(
        _conv1_body,
        grid=(_N // _RT,),
        in_specs=[
            pl.BlockSpec((_K, _RT, 128), lambda t: (0, t, 0)),
            pl.BlockSpec((_RT, 64), lambda t: (t, 0)),
            pl.BlockSpec((64, 64), lambda t: (0, 0)),
            pl.BlockSpec((64,), lambda t: (0,)),
            pl.BlockSpec((64, 128), lambda t: (0, 0)),
            pl.BlockSpec((128,), lambda t: (0,)),
            pl.BlockSpec((64, 128), lambda t: (0, 0)),
        ],
        out_specs=[
            pl.BlockSpec((_RT, 64), lambda t: (t, 0)),
            pl.BlockSpec((_RT, 128), lambda t: (t, 0)),
            pl.BlockSpec((_RT, 128), lambda t: (t, 0)),
        ],
        out_shape=[
            jax.ShapeDtypeStruct((_N, 64), jnp.float32),
            jax.ShapeDtypeStruct((_N, 128), jnp.float32),
            jax.ShapeDtypeStruct((_N, 128), jnp.float32),
        ],
    )(g1.reshape(_K, _N, 128), c1_, b0l1_W, b0l1_b,
      b1l0_W[:64] - b1l0_W[64:], b1l0_b, b1l0_W[64:])

    sq2 = jnp.sum(x1 * x1, axis=1)
    idx2 = _knn(x1, btc2, btr2, sq2, se, cb, 64)
    m2 = _sc_gather_max(d2v, idx2, 128)

    bt3 = batch.reshape(_N // _RT, _RT, 1)
    return pl.pallas_call(
        _aggr_body,
        grid=(_N // _RT,),
        in_specs=[
            pl.BlockSpec((_RT, 64), lambda t: (t, 0)),
            pl.BlockSpec((_RT, 128), lambda t: (t, 0)),
            pl.BlockSpec((_RT, 128), lambda t: (t, 0)),
            pl.BlockSpec((1, _RT, 1), lambda t: (t, 0, 0)),
            pl.BlockSpec((64, 1024), lambda t: (0, 0)),
            pl.BlockSpec((128, 1024), lambda t: (0, 0)),
            pl.BlockSpec((1024,), lambda t: (0,)),
            pl.BlockSpec((1024, 512), lambda t: (0, 0)),
            pl.BlockSpec((512,), lambda t: (0,)),
            pl.BlockSpec((512, 256), lambda t: (0, 0)),
            pl.BlockSpec((256,), lambda t: (0,)),
            pl.BlockSpec((256, 40), lambda t: (0, 0)),
            pl.BlockSpec((40,), lambda t: (0,)),
        ],
        out_specs=pl.BlockSpec((_B, 40), lambda t: (0, 0)),
        out_shape=jax.ShapeDtypeStruct((_B, 40), jnp.float32),
        scratch_shapes=[pltpu.VMEM((_B, 1024), jnp.float32)],
    )(x1, c2, m2, bt3, aggr_W[:64], aggr_W[64:],
      aggr_b, h0_W, h0_b, h1_W, h1_b, h2_W, h2_b)
